# baseline (speedup 1.0000x reference)
import jax
import jax.numpy as jnp
from jax import lax
from jax.experimental import pallas as pl
from jax.experimental.pallas import tpu as pltpu
from jax.experimental.pallas import tpu_sc as plsc

NUM_EMB = 1_000_000
D = 32
S = 8
L = 16
NC, NS = 2, 16
NW = NC * NS
CHUNK = 128
E_TOT = 4096 * 26
PER_W = E_TOT // NW
N_CHUNKS = PER_W // CHUNK


def _transpose_body(tT_ref, o_ref, scr_ref):
    scr_ref[...] = tT_ref[...].T
    for q in range(4):
        o_ref[:, 32 * q:32 * (q + 1)] = scr_ref[q::4, :]


def _row_major_table(table):
    grid = (NUM_EMB + 511) // 512
    packed = pl.pallas_call(
        _transpose_body,
        grid=(grid,),
        in_specs=[pl.BlockSpec((D, 512), lambda j: (jnp.int32(0), j))],
        out_specs=pl.BlockSpec((128, 128), lambda j: (j, jnp.int32(0))),
        out_shape=jax.ShapeDtypeStruct((NUM_EMB * D // 128, 128), jnp.float32),
        scratch_shapes=[pltpu.VMEM((512, D), jnp.float32)],
    )(table.T)
    return packed.reshape(NUM_EMB, D)


def _rsqrt_nr(ss):
    yi = jnp.int32(0x5F3759DF) - (plsc.bitcast(ss, jnp.int32) >> 1)
    y = plsc.bitcast(yi, jnp.float32)
    for _ in range(3):
        y = y * (1.5 - 0.5 * ss * y * y)
    return y


def _body(ids_hbm, table_hbm, out_hbm, ids_v, idx_v, rows_v, sem0, sem1):
    wid = lax.axis_index("s") * NC + lax.axis_index("c")
    base = wid * jnp.int32(PER_W)
    lane = lax.iota(jnp.int32, L)
    sems = (sem0, sem1)

    pltpu.sync_copy(ids_hbm.at[pl.ds(base, PER_W)], ids_v)

    def prep(k, p):
        pi = jnp.int32(p)

        def hash_one(j, c):
            jof = j * jnp.int32(L)
            r = ids_v[pl.ds(k * jnp.int32(CHUNK) + jof, L)]
            r = r % NUM_EMB

            def shift_one(s, rr):
                idx_v[pi, s, pl.ds(jof, L)] = rr
                return (rr << 1) % NUM_EMB

            lax.fori_loop(jnp.int32(0), jnp.int32(S), shift_one, r)
            return c

        lax.fori_loop(jnp.int32(0), jnp.int32(CHUNK // L), hash_one, 0)

        zero = jnp.zeros((L,), jnp.float32)

        def zero_one(e, c):
            rows_v[pi, e, pl.ds(0, L)] = zero
            rows_v[pi, e, pl.ds(L, L)] = zero
            return c

        lax.fori_loop(jnp.int32(0), jnp.int32(CHUNK), zero_one, 0)

        def fire_one(s, c):
            pltpu.async_copy(
                table_hbm.at[idx_v.at[pi, s]],
                rows_v.at[pi], sems[p], add=True)
            return c

        lax.fori_loop(jnp.int32(0), jnp.int32(S), fire_one, 0)

    def consume(k, p):
        pi = jnp.int32(p)

        def drain_one(s, c):
            pltpu.make_async_copy(
                table_hbm.at[idx_v.at[pi, s]],
                rows_v.at[pi], sems[p]).wait()
            return c

        lax.fori_loop(jnp.int32(0), jnp.int32(S), drain_one, 0)

        def norm_group(g, c):
            eidx = g * jnp.int32(L) + lane

            def acc_one(d, ss):
                dd = jnp.zeros((L,), jnp.int32) + d
                v = plsc.load_gather(rows_v.at[pi], [eidx, dd])
                return ss + v * v

            ss = lax.fori_loop(jnp.int32(0), jnp.int32(D), acc_one,
                               jnp.zeros((L,), jnp.float32))
            inv = jnp.minimum(_rsqrt_nr(ss), 1e12)

            def scale_one(d, c2):
                dd = jnp.zeros((L,), jnp.int32) + d
                v = plsc.load_gather(rows_v.at[pi], [eidx, dd])
                plsc.store_scatter(rows_v.at[pi], [eidx, dd], v * inv)
                return c2

            lax.fori_loop(jnp.int32(0), jnp.int32(D), scale_one, 0)
            return c

        lax.fori_loop(jnp.int32(0), jnp.int32(CHUNK // L), norm_group, 0)
        off = base + k * jnp.int32(CHUNK)
        pltpu.sync_copy(rows_v.at[pi], out_hbm.at[pl.ds(off, CHUNK)])

    prep(jnp.int32(0), 0)
    prep(jnp.int32(1), 1)

    def pair(i, c):
        k0 = i * jnp.int32(2)
        consume(k0, 0)

        @pl.when(k0 + 2 < N_CHUNKS)
        def _():
            prep(k0 + jnp.int32(2), 0)

        consume(k0 + jnp.int32(1), 1)

        @pl.when(k0 + 3 < N_CHUNKS)
        def _():
            prep(k0 + jnp.int32(3), 1)

        return c

    lax.fori_loop(jnp.int32(0), jnp.int32(N_CHUNKS // 2), pair, 0)


def kernel(id_, table):
    b, f = id_.shape
    ids32 = id_.reshape(-1).astype(jnp.int32)
    mesh = plsc.VectorSubcoreMesh(
        core_axis_name="c", subcore_axis_name="s",
        num_cores=NC, num_subcores=NS)
    out = pl.kernel(
        _body,
        out_type=jax.ShapeDtypeStruct((E_TOT, D), jnp.float32),
        mesh=mesh,
        compiler_params=pltpu.CompilerParams(
            needs_layout_passes=False, use_tc_tiling_on_sc=False),
        scratch_types=[
            pltpu.VMEM((PER_W,), jnp.int32),
            pltpu.VMEM((2, S, CHUNK), jnp.int32),
            pltpu.VMEM((2, CHUNK, D), jnp.float32),
            pltpu.SemaphoreType.DMA,
            pltpu.SemaphoreType.DMA,
        ],
    )(ids32, _row_major_table(table))
    return out.reshape(b, f, D)


# speedup vs baseline: 2.2492x; 2.2492x over previous
import jax
import jax.numpy as jnp
from jax import lax
from jax.experimental import pallas as pl
from jax.experimental.pallas import tpu as pltpu
from jax.experimental.pallas import tpu_sc as plsc

NUM_EMB = 1_000_000
D = 32
S = 8
L = 16
NC, NS = 2, 16
NW = NC * NS
CHUNK = 128
E_TOT = 4096 * 26
PER_W = E_TOT // NW
N_CHUNKS = PER_W // CHUNK


TBC = 4096


def _transpose_body(tT_ref, o_ref, scr_ref):
    scr_ref[...] = tT_ref[...].T
    for q in range(4):
        o_ref[:, 32 * q:32 * (q + 1)] = scr_ref[q::4, :]


def _row_major_table(table):
    grid = (NUM_EMB + TBC - 1) // TBC
    packed = pl.pallas_call(
        _transpose_body,
        grid=(grid,),
        in_specs=[pl.BlockSpec((D, TBC), lambda j: (jnp.int32(0), j))],
        out_specs=pl.BlockSpec((TBC // 4, 128), lambda j: (j, jnp.int32(0))),
        out_shape=jax.ShapeDtypeStruct((NUM_EMB * D // 128, 128), jnp.float32),
        scratch_shapes=[pltpu.VMEM((TBC, D), jnp.float32)],
    )(table.T)
    return packed.reshape(NUM_EMB, D)


def _rsqrt_nr(ss):
    yi = jnp.int32(0x5F3759DF) - (plsc.bitcast(ss, jnp.int32) >> 1)
    y = plsc.bitcast(yi, jnp.float32)
    for _ in range(3):
        y = y * (1.5 - 0.5 * ss * y * y)
    return y


def _body(ids_hbm, table_hbm, out_hbm, ids_v, idx_v, rows_v, sem0, sem1):
    wid = lax.axis_index("s") * NC + lax.axis_index("c")
    base = wid * jnp.int32(PER_W)
    lane = lax.iota(jnp.int32, L)
    sems = (sem0, sem1)

    pltpu.sync_copy(ids_hbm.at[pl.ds(base, PER_W)], ids_v)

    def prep(k, p):
        pi = jnp.int32(p)

        def hash_one(j, c):
            jof = j * jnp.int32(L)
            r = ids_v[pl.ds(k * jnp.int32(CHUNK) + jof, L)]
            r = r % NUM_EMB

            def shift_one(s, rr):
                idx_v[pi, s, pl.ds(jof, L)] = rr
                return (rr << 1) % NUM_EMB

            lax.fori_loop(jnp.int32(0), jnp.int32(S), shift_one, r)
            return c

        lax.fori_loop(jnp.int32(0), jnp.int32(CHUNK // L), hash_one, 0)

        zero = jnp.zeros((L,), jnp.float32)

        def zero_one(e, c):
            rows_v[pi, e, pl.ds(0, L)] = zero
            rows_v[pi, e, pl.ds(L, L)] = zero
            return c

        lax.fori_loop(jnp.int32(0), jnp.int32(CHUNK), zero_one, 0)

        def fire_one(s, c):
            pltpu.async_copy(
                table_hbm.at[idx_v.at[pi, s]],
                rows_v.at[pi], sems[p], add=True)
            return c

        lax.fori_loop(jnp.int32(0), jnp.int32(S), fire_one, 0)

    def consume(k, p):
        pi = jnp.int32(p)

        def drain_one(s, c):
            pltpu.make_async_copy(
                table_hbm.at[idx_v.at[pi, s]],
                rows_v.at[pi], sems[p]).wait()
            return c

        lax.fori_loop(jnp.int32(0), jnp.int32(S), drain_one, 0)

        def norm_group(g, c):
            eidx = g * jnp.int32(L) + lane

            def acc_one(d, ss):
                dd = jnp.zeros((L,), jnp.int32) + d
                v = plsc.load_gather(rows_v.at[pi], [eidx, dd])
                return ss + v * v

            ss = lax.fori_loop(jnp.int32(0), jnp.int32(D), acc_one,
                               jnp.zeros((L,), jnp.float32))
            inv = jnp.minimum(_rsqrt_nr(ss), 1e12)

            def scale_one(d, c2):
                dd = jnp.zeros((L,), jnp.int32) + d
                v = plsc.load_gather(rows_v.at[pi], [eidx, dd])
                plsc.store_scatter(rows_v.at[pi], [eidx, dd], v * inv)
                return c2

            lax.fori_loop(jnp.int32(0), jnp.int32(D), scale_one, 0)
            return c

        lax.fori_loop(jnp.int32(0), jnp.int32(CHUNK // L), norm_group, 0)
        off = base + k * jnp.int32(CHUNK)
        pltpu.sync_copy(rows_v.at[pi], out_hbm.at[pl.ds(off, CHUNK)])

    prep(jnp.int32(0), 0)
    prep(jnp.int32(1), 1)

    def pair(i, c):
        k0 = i * jnp.int32(2)
        consume(k0, 0)

        @pl.when(k0 + 2 < N_CHUNKS)
        def _():
            prep(k0 + jnp.int32(2), 0)

        consume(k0 + jnp.int32(1), 1)

        @pl.when(k0 + 3 < N_CHUNKS)
        def _():
            prep(k0 + jnp.int32(3), 1)

        return c

    lax.fori_loop(jnp.int32(0), jnp.int32(N_CHUNKS // 2), pair, 0)


def kernel(id_, table):
    b, f = id_.shape
    ids32 = id_.reshape(-1).astype(jnp.int32)
    mesh = plsc.VectorSubcoreMesh(
        core_axis_name="c", subcore_axis_name="s",
        num_cores=NC, num_subcores=NS)
    out = pl.kernel(
        _body,
        out_type=jax.ShapeDtypeStruct((E_TOT, D), jnp.float32),
        mesh=mesh,
        compiler_params=pltpu.CompilerParams(
            needs_layout_passes=False, use_tc_tiling_on_sc=False),
        scratch_types=[
            pltpu.VMEM((PER_W,), jnp.int32),
            pltpu.VMEM((2, S, CHUNK), jnp.int32),
            pltpu.VMEM((2, CHUNK, D), jnp.float32),
            pltpu.SemaphoreType.DMA,
            pltpu.SemaphoreType.DMA,
        ],
    )(ids32, _row_major_table(table))
    return out.reshape(b, f, D)


# speedup vs baseline: 2.3458x; 1.0430x over previous
import jax
import jax.numpy as jnp
from jax import lax
from jax.experimental import pallas as pl
from jax.experimental.pallas import tpu as pltpu
from jax.experimental.pallas import tpu_sc as plsc

NUM_EMB = 1_000_000
D = 32
S = 8
L = 16
NC, NS = 2, 16
NW = NC * NS
CHUNK = 128
E_TOT = 4096 * 26
PER_W = E_TOT // NW
N_CHUNKS = PER_W // CHUNK


TBC = 8192


def _transpose_body(tT_ref, o_ref, scr_ref):
    scr_ref[...] = tT_ref[...].T
    for q in range(4):
        o_ref[:, 32 * q:32 * (q + 1)] = scr_ref[q::4, :]


def _row_major_table(table):
    grid = (NUM_EMB + TBC - 1) // TBC
    packed = pl.pallas_call(
        _transpose_body,
        grid=(grid,),
        in_specs=[pl.BlockSpec((D, TBC), lambda j: (jnp.int32(0), j))],
        out_specs=pl.BlockSpec((TBC // 4, 128), lambda j: (j, jnp.int32(0))),
        out_shape=jax.ShapeDtypeStruct((NUM_EMB * D // 128, 128), jnp.float32),
        scratch_shapes=[pltpu.VMEM((TBC, D), jnp.float32)],
    )(table.T)
    return packed.reshape(NUM_EMB, D)


def _rsqrt_nr(ss):
    yi = jnp.int32(0x5F3759DF) - (plsc.bitcast(ss, jnp.int32) >> 1)
    y = plsc.bitcast(yi, jnp.float32)
    for _ in range(3):
        y = y * (1.5 - 0.5 * ss * y * y)
    return y


def _body(ids_hbm, table_hbm, out_hbm, ids_v, idx_v, rows_v, sem0, sem1):
    wid = lax.axis_index("s") * NC + lax.axis_index("c")
    base = wid * jnp.int32(PER_W)
    lane = lax.iota(jnp.int32, L)
    sems = (sem0, sem1)

    pltpu.sync_copy(ids_hbm.at[pl.ds(base, PER_W)], ids_v)

    def prep(k, p):
        pi = jnp.int32(p)

        def hash_one(j, c):
            jof = j * jnp.int32(L)
            r = ids_v[pl.ds(k * jnp.int32(CHUNK) + jof, L)]
            r = r % NUM_EMB

            def shift_one(s, rr):
                idx_v[pi, s, pl.ds(jof, L)] = rr
                return (rr << 1) % NUM_EMB

            lax.fori_loop(jnp.int32(0), jnp.int32(S), shift_one, r)
            return c

        lax.fori_loop(jnp.int32(0), jnp.int32(CHUNK // L), hash_one, 0)

        zero = jnp.zeros((L,), jnp.float32)

        def zero_one(e, c):
            rows_v[pi, e, pl.ds(0, L)] = zero
            rows_v[pi, e, pl.ds(L, L)] = zero
            return c

        lax.fori_loop(jnp.int32(0), jnp.int32(CHUNK), zero_one, 0)

        def fire_one(s, c):
            pltpu.async_copy(
                table_hbm.at[idx_v.at[pi, s]],
                rows_v.at[pi], sems[p], add=True)
            return c

        lax.fori_loop(jnp.int32(0), jnp.int32(S), fire_one, 0)

    def consume(k, p):
        pi = jnp.int32(p)

        def drain_one(s, c):
            pltpu.make_async_copy(
                table_hbm.at[idx_v.at[pi, s]],
                rows_v.at[pi], sems[p]).wait()
            return c

        lax.fori_loop(jnp.int32(0), jnp.int32(S), drain_one, 0)

        def norm_group(g, c):
            eidx = g * jnp.int32(L) + lane

            def acc_one(d, ss):
                dd = jnp.zeros((L,), jnp.int32) + d
                v = plsc.load_gather(rows_v.at[pi], [eidx, dd])
                return ss + v * v

            ss = lax.fori_loop(jnp.int32(0), jnp.int32(D), acc_one,
                               jnp.zeros((L,), jnp.float32))
            inv = jnp.minimum(_rsqrt_nr(ss), 1e12)

            def scale_one(d, c2):
                dd = jnp.zeros((L,), jnp.int32) + d
                v = plsc.load_gather(rows_v.at[pi], [eidx, dd])
                plsc.store_scatter(rows_v.at[pi], [eidx, dd], v * inv)
                return c2

            lax.fori_loop(jnp.int32(0), jnp.int32(D), scale_one, 0)
            return c

        lax.fori_loop(jnp.int32(0), jnp.int32(CHUNK // L), norm_group, 0)
        off = base + k * jnp.int32(CHUNK)
        pltpu.sync_copy(rows_v.at[pi], out_hbm.at[pl.ds(off, CHUNK)])

    prep(jnp.int32(0), 0)
    prep(jnp.int32(1), 1)

    def pair(i, c):
        k0 = i * jnp.int32(2)
        consume(k0, 0)

        @pl.when(k0 + 2 < N_CHUNKS)
        def _():
            prep(k0 + jnp.int32(2), 0)

        consume(k0 + jnp.int32(1), 1)

        @pl.when(k0 + 3 < N_CHUNKS)
        def _():
            prep(k0 + jnp.int32(3), 1)

        return c

    lax.fori_loop(jnp.int32(0), jnp.int32(N_CHUNKS // 2), pair, 0)


def kernel(id_, table):
    b, f = id_.shape
    ids32 = id_.reshape(-1).astype(jnp.int32)
    mesh = plsc.VectorSubcoreMesh(
        core_axis_name="c", subcore_axis_name="s",
        num_cores=NC, num_subcores=NS)
    out = pl.kernel(
        _body,
        out_type=jax.ShapeDtypeStruct((E_TOT, D), jnp.float32),
        mesh=mesh,
        compiler_params=pltpu.CompilerParams(
            needs_layout_passes=False, use_tc_tiling_on_sc=False),
        scratch_types=[
            pltpu.VMEM((PER_W,), jnp.int32),
            pltpu.VMEM((2, S, CHUNK), jnp.int32),
            pltpu.VMEM((2, CHUNK, D), jnp.float32),
            pltpu.SemaphoreType.DMA,
            pltpu.SemaphoreType.DMA,
        ],
    )(ids32, _row_major_table(table))
    return out.reshape(b, f, D)


# speedup vs baseline: 3.2326x; 1.3780x over previous
import jax
import jax.numpy as jnp
from jax import lax
from jax.experimental import pallas as pl
from jax.experimental.pallas import tpu as pltpu
from jax.experimental.pallas import tpu_sc as plsc

NUM_EMB = 1_000_000
D = 32
S = 8
L = 16
NC, NS = 2, 16
NW = NC * NS
CHUNK = 128
E_TOT = 4096 * 26
PER_W = E_TOT // NW
N_CHUNKS = PER_W // CHUNK


TBC = 8192


def _transpose_body(tT_ref, o_ref, scr_ref):
    scr_ref[...] = tT_ref[...].T
    for q in range(4):
        o_ref[:, 32 * q:32 * (q + 1)] = scr_ref[q::4, :]


def _row_major_table(table):
    grid = (NUM_EMB + TBC - 1) // TBC
    packed = pl.pallas_call(
        _transpose_body,
        grid=(grid,),
        in_specs=[pl.BlockSpec((D, TBC), lambda j: (jnp.int32(0), j))],
        out_specs=pl.BlockSpec((TBC // 4, 128), lambda j: (j, jnp.int32(0))),
        out_shape=jax.ShapeDtypeStruct((NUM_EMB * D // 128, 128), jnp.float32),
        scratch_shapes=[pltpu.VMEM((TBC, D), jnp.float32)],
    )(table.T)
    return packed.reshape(NUM_EMB, D)


NBR = 2048


def _norm_body(x_ref, o_ref):
    x = x_ref[...]
    r = lax.broadcasted_iota(jnp.int32, (128, 128), 0) // D
    c = lax.broadcasted_iota(jnp.int32, (128, 128), 1) // D
    blk = (r == c).astype(jnp.float32)
    ss = lax.dot(x * x, blk, precision=lax.Precision.HIGHEST)
    o_ref[...] = x / jnp.maximum(jnp.sqrt(ss), 1e-12)


def _normalize(flat):
    rows = flat.shape[0]
    return pl.pallas_call(
        _norm_body,
        grid=(rows // NBR,),
        in_specs=[pl.BlockSpec((NBR, 128), lambda j: (j, jnp.int32(0)))],
        out_specs=pl.BlockSpec((NBR, 128), lambda j: (j, jnp.int32(0))),
        out_shape=jax.ShapeDtypeStruct((rows, 128), jnp.float32),
    )(flat)


def _body(ids_hbm, table_hbm, out_hbm, ids_v, idx_v, rows_v, sem0, sem1):
    wid = lax.axis_index("s") * NC + lax.axis_index("c")
    base = wid * jnp.int32(PER_W)
    sems = (sem0, sem1)

    pltpu.sync_copy(ids_hbm.at[pl.ds(base, PER_W)], ids_v)

    def prep(k, p):
        pi = jnp.int32(p)

        def hash_one(j, c):
            jof = j * jnp.int32(L)
            r = ids_v[pl.ds(k * jnp.int32(CHUNK) + jof, L)]
            r = r % NUM_EMB

            def shift_one(s, rr):
                idx_v[pi, s, pl.ds(jof, L)] = rr
                return (rr << 1) % NUM_EMB

            lax.fori_loop(jnp.int32(0), jnp.int32(S), shift_one, r)
            return c

        lax.fori_loop(jnp.int32(0), jnp.int32(CHUNK // L), hash_one, 0)

        zero = jnp.zeros((L,), jnp.float32)

        def zero_one(e, c):
            rows_v[pi, e, pl.ds(0, L)] = zero
            rows_v[pi, e, pl.ds(L, L)] = zero
            return c

        lax.fori_loop(jnp.int32(0), jnp.int32(CHUNK), zero_one, 0)

        def fire_one(s, c):
            pltpu.async_copy(
                table_hbm.at[idx_v.at[pi, s]],
                rows_v.at[pi], sems[p], add=True)
            return c

        lax.fori_loop(jnp.int32(0), jnp.int32(S), fire_one, 0)

    def consume(k, p):
        pi = jnp.int32(p)

        def drain_one(s, c):
            pltpu.make_async_copy(
                table_hbm.at[idx_v.at[pi, s]],
                rows_v.at[pi], sems[p]).wait()
            return c

        lax.fori_loop(jnp.int32(0), jnp.int32(S), drain_one, 0)
        off = base + k * jnp.int32(CHUNK)
        pltpu.sync_copy(rows_v.at[pi], out_hbm.at[pl.ds(off, CHUNK)])

    prep(jnp.int32(0), 0)
    prep(jnp.int32(1), 1)

    def pair(i, c):
        k0 = i * jnp.int32(2)
        consume(k0, 0)

        @pl.when(k0 + 2 < N_CHUNKS)
        def _():
            prep(k0 + jnp.int32(2), 0)

        consume(k0 + jnp.int32(1), 1)

        @pl.when(k0 + 3 < N_CHUNKS)
        def _():
            prep(k0 + jnp.int32(3), 1)

        return c

    lax.fori_loop(jnp.int32(0), jnp.int32(N_CHUNKS // 2), pair, 0)


def kernel(id_, table):
    b, f = id_.shape
    ids32 = id_.reshape(-1).astype(jnp.int32)
    mesh = plsc.VectorSubcoreMesh(
        core_axis_name="c", subcore_axis_name="s",
        num_cores=NC, num_subcores=NS)
    out = pl.kernel(
        _body,
        out_type=jax.ShapeDtypeStruct((E_TOT, D), jnp.float32),
        mesh=mesh,
        compiler_params=pltpu.CompilerParams(
            needs_layout_passes=False, use_tc_tiling_on_sc=False),
        scratch_types=[
            pltpu.VMEM((PER_W,), jnp.int32),
            pltpu.VMEM((2, S, CHUNK), jnp.int32),
            pltpu.VMEM((2, CHUNK, D), jnp.float32),
            pltpu.SemaphoreType.DMA,
            pltpu.SemaphoreType.DMA,
        ],
    )(ids32, _row_major_table(table))
    normed = _normalize(out.reshape(E_TOT * D // 128, 128))
    return normed.reshape(b, f, D)


# speedup vs baseline: 3.2579x; 1.0078x over previous
import jax
import jax.numpy as jnp
from jax import lax
from jax.experimental import pallas as pl
from jax.experimental.pallas import tpu as pltpu
from jax.experimental.pallas import tpu_sc as plsc

NUM_EMB = 1_000_000
D = 32
S = 8
L = 16
NC, NS = 2, 16
NW = NC * NS
CHUNK = 128
E_TOT = 4096 * 26
PER_W = E_TOT // NW
N_CHUNKS = PER_W // CHUNK


TBC = 16384


def _transpose_body(tT_ref, o_ref, scr_ref):
    scr_ref[...] = tT_ref[...].T
    for q in range(4):
        o_ref[:, 32 * q:32 * (q + 1)] = scr_ref[q::4, :]


def _row_major_table(table):
    grid = (NUM_EMB + TBC - 1) // TBC
    packed = pl.pallas_call(
        _transpose_body,
        grid=(grid,),
        in_specs=[pl.BlockSpec((D, TBC), lambda j: (jnp.int32(0), j))],
        out_specs=pl.BlockSpec((TBC // 4, 128), lambda j: (j, jnp.int32(0))),
        out_shape=jax.ShapeDtypeStruct((NUM_EMB * D // 128, 128), jnp.float32),
        scratch_shapes=[pltpu.VMEM((TBC, D), jnp.float32)],
    )(table.T)
    return packed.reshape(NUM_EMB, D)


NBR = 2048


def _norm_body(x_ref, o_ref):
    x = x_ref[...]
    r = lax.broadcasted_iota(jnp.int32, (128, 128), 0) // D
    c = lax.broadcasted_iota(jnp.int32, (128, 128), 1) // D
    blk = (r == c).astype(jnp.float32)
    ss = lax.dot(x * x, blk, precision=lax.Precision.HIGHEST)
    o_ref[...] = x / jnp.maximum(jnp.sqrt(ss), 1e-12)


def _normalize(flat):
    rows = flat.shape[0]
    return pl.pallas_call(
        _norm_body,
        grid=(rows // NBR,),
        in_specs=[pl.BlockSpec((NBR, 128), lambda j: (j, jnp.int32(0)))],
        out_specs=pl.BlockSpec((NBR, 128), lambda j: (j, jnp.int32(0))),
        out_shape=jax.ShapeDtypeStruct((rows, 128), jnp.float32),
    )(flat)


def _body(ids_hbm, table_hbm, out_hbm, ids_v, idx_v, rows_v, sem0, sem1):
    wid = lax.axis_index("s") * NC + lax.axis_index("c")
    base = wid * jnp.int32(PER_W)
    sems = (sem0, sem1)

    pltpu.sync_copy(ids_hbm.at[pl.ds(base, PER_W)], ids_v)

    def prep(k, p):
        pi = jnp.int32(p)

        def hash_one(j, c):
            jof = j * jnp.int32(L)
            r = ids_v[pl.ds(k * jnp.int32(CHUNK) + jof, L)]
            r = r % NUM_EMB

            def shift_one(s, rr):
                idx_v[pi, s, pl.ds(jof, L)] = rr
                return (rr << 1) % NUM_EMB

            lax.fori_loop(jnp.int32(0), jnp.int32(S), shift_one, r)
            return c

        lax.fori_loop(jnp.int32(0), jnp.int32(CHUNK // L), hash_one, 0)

        zero = jnp.zeros((L,), jnp.float32)

        def zero_one(e, c):
            rows_v[pi, e, pl.ds(0, L)] = zero
            rows_v[pi, e, pl.ds(L, L)] = zero
            return c

        lax.fori_loop(jnp.int32(0), jnp.int32(CHUNK), zero_one, 0)

        def fire_one(s, c):
            pltpu.async_copy(
                table_hbm.at[idx_v.at[pi, s]],
                rows_v.at[pi], sems[p], add=True)
            return c

        lax.fori_loop(jnp.int32(0), jnp.int32(S), fire_one, 0)

    def consume(k, p):
        pi = jnp.int32(p)

        def drain_one(s, c):
            pltpu.make_async_copy(
                table_hbm.at[idx_v.at[pi, s]],
                rows_v.at[pi], sems[p]).wait()
            return c

        lax.fori_loop(jnp.int32(0), jnp.int32(S), drain_one, 0)
        off = base + k * jnp.int32(CHUNK)
        pltpu.sync_copy(rows_v.at[pi], out_hbm.at[pl.ds(off, CHUNK)])

    prep(jnp.int32(0), 0)
    prep(jnp.int32(1), 1)

    def pair(i, c):
        k0 = i * jnp.int32(2)
        consume(k0, 0)

        @pl.when(k0 + 2 < N_CHUNKS)
        def _():
            prep(k0 + jnp.int32(2), 0)

        consume(k0 + jnp.int32(1), 1)

        @pl.when(k0 + 3 < N_CHUNKS)
        def _():
            prep(k0 + jnp.int32(3), 1)

        return c

    lax.fori_loop(jnp.int32(0), jnp.int32(N_CHUNKS // 2), pair, 0)


def kernel(id_, table):
    b, f = id_.shape
    ids32 = id_.reshape(-1).astype(jnp.int32)
    mesh = plsc.VectorSubcoreMesh(
        core_axis_name="c", subcore_axis_name="s",
        num_cores=NC, num_subcores=NS)
    out = pl.kernel(
        _body,
        out_type=jax.ShapeDtypeStruct((E_TOT, D), jnp.float32),
        mesh=mesh,
        compiler_params=pltpu.CompilerParams(
            needs_layout_passes=False, use_tc_tiling_on_sc=False),
        scratch_types=[
            pltpu.VMEM((PER_W,), jnp.int32),
            pltpu.VMEM((2, S, CHUNK), jnp.int32),
            pltpu.VMEM((2, CHUNK, D), jnp.float32),
            pltpu.SemaphoreType.DMA,
            pltpu.SemaphoreType.DMA,
        ],
    )(ids32, _row_major_table(table))
    normed = _normalize(out.reshape(E_TOT * D // 128, 128))
    return normed.reshape(b, f, D)
